# baseline (speedup 1.0000x reference)
import functools

import jax
import jax.numpy as jnp
from jax import lax
from jax.experimental import pallas as pl
from jax.experimental.pallas import tpu as pltpu
from jax.experimental.pallas import tpu_sc as plsc

_B = 16384
_H = 200
_D = 32
_NUM_CORES = 2
_NUM_SUBCORES = 16
_NUM_WORKERS = _NUM_CORES * _NUM_SUBCORES
_BW = _B // _NUM_WORKERS


def _make_gather(vocab):
    mesh = plsc.VectorSubcoreMesh(core_axis_name="c", subcore_axis_name="s")

    @functools.partial(
        pl.kernel,
        mesh=mesh,
        out_type=jax.ShapeDtypeStruct((_H, _D // 8, _B // 128, 8, 128), jnp.float32),
        compiler_params=pltpu.CompilerParams(use_tc_tiling_on_sc=False, needs_layout_passes=False),
        scratch_types=[
            pltpu.VMEM((_BW,), jnp.int32),
            pltpu.VMEM((_BW, _D), jnp.float32),
            pltpu.VMEM((_D // 8, _BW // 128, 8, 128), jnp.float32),
            pltpu.SemaphoreType.DMA,
        ],
    )
    def gather_kernel(ids_hbm, table_hbm, out_hbm, idx_v, rows_v, tiles_v, sem):
        w = lax.axis_index("s") * _NUM_CORES + lax.axis_index("c")
        b0 = w * _BW
        blk0 = w * (_BW // 128)
        lane = jnp.arange(16, dtype=jnp.int32)

        def step(h, _):
            pltpu.sync_copy(ids_hbm.at[pl.ds(h * _B + b0, _BW)], idx_v)
            pltpu.async_copy(table_hbm.at[idx_v], rows_v, sem).wait()

            def xpose(g, _):
                row_idx = g * 16 + lane
                blk = g // 8
                sub = g % 8
                for e in range(_D):
                    v = plsc.load_gather(rows_v, [row_idx, jnp.full((16,), e, jnp.int32)])
                    tiles_v[e // 8, blk, e % 8, pl.ds(sub * 16, 16)] = v
                return 0

            lax.fori_loop(0, _BW // 16, xpose, 0)
            pltpu.sync_copy(tiles_v, out_hbm.at[h, :, pl.ds(blk0, _BW // 128)])
            return 0

        lax.fori_loop(0, _H, step, 0)

    return gather_kernel


def kernel(ids, embeddings):
    batch, hist = ids.shape
    ids_lin = ids.T.reshape(batch * hist).astype(jnp.int32)
    out5 = _make_gather(embeddings.shape[0])(ids_lin, embeddings)
    out = out5.transpose(2, 4, 0, 1, 3).reshape(batch, hist, _D)
    return out


# speedup vs baseline: 1.6697x; 1.6697x over previous
import functools

import jax
import jax.numpy as jnp
from jax import lax
from jax.experimental import pallas as pl
from jax.experimental.pallas import tpu as pltpu
from jax.experimental.pallas import tpu_sc as plsc

_B = 16384
_H = 200
_D = 32
_NUM_CORES = 2
_NUM_SUBCORES = 16
_NUM_WORKERS = _NUM_CORES * _NUM_SUBCORES
_BW = _B // _NUM_WORKERS
_NB = _BW // 128
_JB = 8
_NBLK = _H // _JB


def _make_gather(vocab):
    mesh = plsc.VectorSubcoreMesh(core_axis_name="c", subcore_axis_name="s")

    @functools.partial(
        pl.kernel,
        mesh=mesh,
        out_type=jax.ShapeDtypeStruct((_H, _D // 8, _B // 128, 8, 128), jnp.float32),
        compiler_params=pltpu.CompilerParams(
            use_tc_tiling_on_sc=False, needs_layout_passes=False
        ),
        scratch_types=[
            pltpu.VMEM((2, _JB, _BW), jnp.int32),
            pltpu.VMEM((2 * _BW, _D), jnp.float32),
            pltpu.VMEM((2, _D // 8, _NB, 8, 128), jnp.float32),
            pltpu.SemaphoreType.DMA((2,)),
            pltpu.SemaphoreType.DMA((2,)),
            pltpu.SemaphoreType.DMA((2,)),
        ],
    )
    def gather_kernel(
        ids_hbm, table_hbm, out_hbm, idx_v, rows_v, tiles_v, sem_i, sem_g, sem_s
    ):
        w = lax.axis_index("s") * _NUM_CORES + lax.axis_index("c")
        b0 = w * _BW
        blk0 = w * _NB
        lane = jnp.arange(16, dtype=jnp.int32)

        def idx_copy(blk_i, buf):
            return pltpu.make_async_copy(
                ids_hbm.at[pl.ds(blk_i * _JB, _JB), pl.ds(b0, _BW)],
                idx_v.at[buf],
                sem_i.at[buf],
            )

        def gather_copy(buf, j, cur):
            return pltpu.make_async_copy(
                table_hbm.at[idx_v.at[buf, j]],
                rows_v.at[pl.ds(cur * _BW, _BW)],
                sem_g.at[cur],
            )

        def scatter_copy(h, cur):
            return pltpu.make_async_copy(
                tiles_v.at[cur],
                out_hbm.at[h, :, pl.ds(blk0, _NB)],
                sem_s.at[cur],
            )

        idx_copy(0, 0).start()
        idx_copy(0, 0).wait()
        gather_copy(0, 0, 0).start()

        def block(hb, _):
            bbuf = lax.rem(hb, 2)
            nbuf = lax.rem(hb + 1, 2)

            @pl.when(hb < _NBLK - 1)
            def _():
                idx_copy(hb + 1, nbuf).start()

            for j in range(_JB):
                h = hb * _JB + j
                cur = j % 2
                nxt = (j + 1) % 2

                if j < _JB - 1:
                    gather_copy(bbuf, j + 1, nxt).start()
                else:

                    @pl.when(hb < _NBLK - 1)
                    def _():
                        idx_copy(hb + 1, nbuf).wait()
                        gather_copy(nbuf, 0, nxt).start()

                gather_copy(bbuf, j, cur).wait()

                if j >= 2:
                    scatter_copy(h, cur).wait()
                else:

                    @pl.when(hb > 0)
                    def _():
                        scatter_copy(h, cur).wait()

                def xpose(g, _):
                    row_idx = cur * _BW + g * 16 + lane
                    blk = jnp.right_shift(g, 3)
                    s16 = jnp.bitwise_and(g, 7) * 16
                    for eq in range(_D // 4):
                        es = range(eq * 4, eq * 4 + 4)
                        vs = [
                            plsc.load_gather(
                                rows_v, [row_idx, jnp.full((16,), e, jnp.int32)]
                            )
                            for e in es
                        ]
                        for e, v in zip(es, vs):
                            tiles_v[cur, e // 8, blk, e % 8, pl.ds(s16, 16)] = v
                    return 0

                lax.fori_loop(0, _BW // 16, xpose, 0)
                scatter_copy(h, cur).start()
            return 0

        lax.fori_loop(0, _NBLK, block, 0)
        scatter_copy(_H - 2, 0).wait()
        scatter_copy(_H - 1, 1).wait()

    return gather_kernel


def kernel(ids, embeddings):
    batch, hist = ids.shape
    ids_t = ids.T.astype(jnp.int32)
    out5 = _make_gather(embeddings.shape[0])(ids_t, embeddings)
    out = out5.transpose(2, 4, 0, 1, 3).reshape(batch, hist, _D)
    return out


# speedup vs baseline: 2.6918x; 1.6121x over previous
import functools

import jax
import jax.numpy as jnp
from jax import lax
from jax.experimental import pallas as pl
from jax.experimental.pallas import tpu as pltpu
from jax.experimental.pallas import tpu_sc as plsc

_B = 16384
_H = 200
_D = 32
_NUM_CORES = 2
_NUM_SUBCORES = 16
_NUM_WORKERS = _NUM_CORES * _NUM_SUBCORES
_BW = _B // _NUM_WORKERS
_NB = _BW // 128
_JB = 8
_NBLK = _H // _JB


def _make_gather(vocab):
    mesh = plsc.VectorSubcoreMesh(core_axis_name="c", subcore_axis_name="s")

    @functools.partial(
        pl.kernel,
        mesh=mesh,
        out_type=jax.ShapeDtypeStruct((_H, _D // 8, _B // 128, 8, 128), jnp.float32),
        compiler_params=pltpu.CompilerParams(
            use_tc_tiling_on_sc=False, needs_layout_passes=False
        ),
        scratch_types=[
            pltpu.VMEM((2, _JB, _BW), jnp.int32),
            pltpu.VMEM((2 * _BW, _D), jnp.float32),
            pltpu.VMEM((_BW, _D + 1), jnp.float32),
            pltpu.VMEM((2, _D // 8, _NB, 8, 128), jnp.float32),
            pltpu.SemaphoreType.DMA((2,)),
            pltpu.SemaphoreType.DMA((2,)),
            pltpu.SemaphoreType.DMA((2,)),
        ],
    )
    def gather_kernel(
        ids_hbm, table_hbm, out_hbm, idx_v, rows_v, rows_s, tiles_v, sem_i, sem_g, sem_s
    ):
        w = lax.axis_index("s") * _NUM_CORES + lax.axis_index("c")
        b0 = w * _BW
        blk0 = w * _NB
        lane = jnp.arange(16, dtype=jnp.int32)

        def idx_copy(blk_i, buf):
            return pltpu.make_async_copy(
                ids_hbm.at[pl.ds(blk_i * _JB, _JB), pl.ds(b0, _BW)],
                idx_v.at[buf],
                sem_i.at[buf],
            )

        def gather_copy(buf, j, cur):
            return pltpu.make_async_copy(
                table_hbm.at[idx_v.at[buf, j]],
                rows_v.at[pl.ds(cur * _BW, _BW)],
                sem_g.at[cur],
            )

        def scatter_copy(h, cur):
            return pltpu.make_async_copy(
                tiles_v.at[cur],
                out_hbm.at[h, :, pl.ds(blk0, _NB)],
                sem_s.at[cur],
            )

        idx_copy(0, 0).start()
        idx_copy(0, 0).wait()
        gather_copy(0, 0, 0).start()

        def block(hb, _):
            bbuf = lax.rem(hb, 2)
            nbuf = lax.rem(hb + 1, 2)

            @pl.when(hb < _NBLK - 1)
            def _():
                idx_copy(hb + 1, nbuf).start()

            for j in range(_JB):
                h = hb * _JB + j
                cur = j % 2
                nxt = (j + 1) % 2

                if j < _JB - 1:
                    gather_copy(bbuf, j + 1, nxt).start()
                else:

                    @pl.when(hb < _NBLK - 1)
                    def _():
                        idx_copy(hb + 1, nbuf).wait()
                        gather_copy(nbuf, 0, nxt).start()

                gather_copy(bbuf, j, cur).wait()

                if j >= 2:
                    scatter_copy(h, cur).wait()
                else:

                    @pl.when(hb > 0)
                    def _():
                        scatter_copy(h, cur).wait()

                def swz(rq, _):
                    for k in range(4):
                        r = rq * 4 + k
                        a = rows_v[cur * _BW + r, pl.ds(0, 16)]
                        b = rows_v[cur * _BW + r, pl.ds(16, 16)]
                        rows_s[r, pl.ds(0, 16)] = a
                        rows_s[r, pl.ds(16, 16)] = b
                    return 0

                lax.fori_loop(0, _BW // 4, swz, 0)

                def xpose(g, _):
                    row_idx = g * 16 + lane
                    blk = jnp.right_shift(g, 3)
                    s16 = jnp.bitwise_and(g, 7) * 16
                    for eq in range(_D // 4):
                        es = range(eq * 4, eq * 4 + 4)
                        vs = [
                            plsc.load_gather(
                                rows_s, [row_idx, jnp.full((16,), e, jnp.int32)]
                            )
                            for e in es
                        ]
                        for e, v in zip(es, vs):
                            tiles_v[cur, e // 8, blk, e % 8, pl.ds(s16, 16)] = v
                    return 0

                lax.fori_loop(0, _BW // 16, xpose, 0)
                scatter_copy(h, cur).start()
            return 0

        lax.fori_loop(0, _NBLK, block, 0)
        scatter_copy(_H - 2, 0).wait()
        scatter_copy(_H - 1, 1).wait()

    return gather_kernel


def kernel(ids, embeddings):
    batch, hist = ids.shape
    ids_t = ids.T.astype(jnp.int32)
    out5 = _make_gather(embeddings.shape[0])(ids_t, embeddings)
    out = out5.transpose(2, 4, 0, 1, 3).reshape(batch, hist, _D)
    return out


# speedup vs baseline: 2.6931x; 1.0005x over previous
import functools

import jax
import jax.numpy as jnp
from jax import lax
from jax.experimental import pallas as pl
from jax.experimental.pallas import tpu as pltpu
from jax.experimental.pallas import tpu_sc as plsc

_B = 16384
_H = 200
_D = 32
_NUM_CORES = 2
_NUM_SUBCORES = 16
_NUM_WORKERS = _NUM_CORES * _NUM_SUBCORES
_BW = _B // _NUM_WORKERS
_NB = _BW // 128
_JB = 8
_NBLK = _H // _JB


def _make_gather(vocab):
    mesh = plsc.VectorSubcoreMesh(core_axis_name="c", subcore_axis_name="s")

    @functools.partial(
        pl.kernel,
        mesh=mesh,
        out_type=jax.ShapeDtypeStruct((_H, _D // 8, _B // 128, 8, 128), jnp.float32),
        compiler_params=pltpu.CompilerParams(
            use_tc_tiling_on_sc=False, needs_layout_passes=False
        ),
        scratch_types=[
            pltpu.VMEM((2, _JB, _BW), jnp.int32),
            pltpu.VMEM((4 * _BW, _D), jnp.float32),
            pltpu.VMEM((_BW, _D + 1), jnp.float32),
            pltpu.VMEM((2, _D // 8, _NB, 8, 128), jnp.float32),
            pltpu.SemaphoreType.DMA((2,)),
            pltpu.SemaphoreType.DMA((4,)),
            pltpu.SemaphoreType.DMA((2,)),
        ],
    )
    def gather_kernel(
        ids_hbm, table_hbm, out_hbm, idx_v, rows_v, rows_s, tiles_v, sem_i, sem_g, sem_s
    ):
        w = lax.axis_index("s") * _NUM_CORES + lax.axis_index("c")
        b0 = w * _BW
        blk0 = w * _NB
        lane = jnp.arange(16, dtype=jnp.int32)

        def idx_copy(blk_i, buf):
            return pltpu.make_async_copy(
                ids_hbm.at[pl.ds(blk_i * _JB, _JB), pl.ds(b0, _BW)],
                idx_v.at[buf],
                sem_i.at[buf],
            )

        def gather_copy(buf, j, cur):
            return pltpu.make_async_copy(
                table_hbm.at[idx_v.at[buf, j]],
                rows_v.at[pl.ds(cur * _BW, _BW)],
                sem_g.at[cur],
            )

        def scatter_copy(h, cur):
            return pltpu.make_async_copy(
                tiles_v.at[cur],
                out_hbm.at[h, :, pl.ds(blk0, _NB)],
                sem_s.at[cur],
            )

        idx_copy(0, 0).start()
        idx_copy(0, 0).wait()
        gather_copy(0, 0, 0).start()
        gather_copy(0, 1, 1).start()

        def block(hb, _):
            bbuf = lax.rem(hb, 2)
            nbuf = lax.rem(hb + 1, 2)

            @pl.when(hb < _NBLK - 1)
            def _():
                idx_copy(hb + 1, nbuf).start()

            for j in range(_JB):
                h = hb * _JB + j
                cur = j % 2
                cur4 = j % 4
                nx2 = (j + 2) % 4

                if j < _JB - 2:
                    gather_copy(bbuf, j + 2, nx2).start()
                elif j == _JB - 2:

                    @pl.when(hb < _NBLK - 1)
                    def _():
                        idx_copy(hb + 1, nbuf).wait()
                        gather_copy(nbuf, 0, nx2).start()
                else:

                    @pl.when(hb < _NBLK - 1)
                    def _():
                        gather_copy(nbuf, 1, nx2).start()

                gather_copy(bbuf, j, cur4).wait()

                if j >= 2:
                    scatter_copy(h, cur).wait()
                else:

                    @pl.when(hb > 0)
                    def _():
                        scatter_copy(h, cur).wait()

                def swz(rq, _):
                    for k in range(4):
                        r = rq * 4 + k
                        a = rows_v[cur4 * _BW + r, pl.ds(0, 16)]
                        b = rows_v[cur4 * _BW + r, pl.ds(16, 16)]
                        rows_s[r, pl.ds(0, 16)] = a
                        rows_s[r, pl.ds(16, 16)] = b
                    return 0

                lax.fori_loop(0, _BW // 4, swz, 0)

                def xpose(g, _):
                    row_idx = g * 16 + lane
                    blk = jnp.right_shift(g, 3)
                    s16 = jnp.bitwise_and(g, 7) * 16
                    for eq in range(_D // 4):
                        es = range(eq * 4, eq * 4 + 4)
                        vs = [
                            plsc.load_gather(
                                rows_s, [row_idx, jnp.full((16,), e, jnp.int32)]
                            )
                            for e in es
                        ]
                        for e, v in zip(es, vs):
                            tiles_v[cur, e // 8, blk, e % 8, pl.ds(s16, 16)] = v
                    return 0

                lax.fori_loop(0, _BW // 16, xpose, 0)
                scatter_copy(h, cur).start()
            return 0

        lax.fori_loop(0, _NBLK, block, 0)
        scatter_copy(_H - 2, 0).wait()
        scatter_copy(_H - 1, 1).wait()

    return gather_kernel


def kernel(ids, embeddings):
    batch, hist = ids.shape
    ids_t = ids.T.astype(jnp.int32)
    out5 = _make_gather(embeddings.shape[0])(ids_t, embeddings)
    out = out5.transpose(2, 4, 0, 1, 3).reshape(batch, hist, _D)
    return out
